# Initial kernel scaffold; baseline (speedup 1.0000x reference)
import jax
import jax.numpy as jnp
from jax.experimental import pallas as pl


def kernel(preds, targets):
    raise NotImplementedError("write your pallas kernel here")


# speedup vs baseline: 14.8540x; 14.8540x over previous
import functools

import jax
import jax.numpy as jnp
from jax import lax
from jax.experimental import pallas as pl
from jax.experimental.pallas import tpu as pltpu
from jax.experimental.pallas import tpu_sc as plsc

NC = 2
NS = 16
L = 16
B = 8
N = 512 * 512
IPC = B // NC
CHUNK = N // NS
NBINS = 1024
EMAX = 16.0
SCALE = NBINS / EMAX
HW = L * 4 * NBINS
RW = 4 * NBINS
BR = NBINS // NS
BLK = 4 * BR

_mesh = plsc.VectorSubcoreMesh(
    core_axis_name="c", subcore_axis_name="s", num_cores=NC, num_subcores=NS)


@functools.partial(
    pl.kernel,
    out_type=jax.ShapeDtypeStruct((NC, L), jnp.float32),
    mesh=_mesh,
    scratch_types=[
        pltpu.VMEM((CHUNK,), jnp.float32),
        pltpu.VMEM((CHUNK,), jnp.int32),
        pltpu.VMEM((HW,), jnp.float32),
        pltpu.VMEM((RW,), jnp.float32),
        pltpu.VMEM((BLK,), jnp.float32),
        pltpu.VMEM((L,), jnp.float32),
        pltpu.VMEM_SHARED((NS * BLK * NS,), jnp.float32),
        pltpu.VMEM_SHARED((NS * L,), jnp.float32),
        pltpu.VMEM_SHARED((NS * L,), jnp.float32),
        pltpu.VMEM_SHARED((NS * L,), jnp.float32),
        pltpu.VMEM_SHARED((NS * L,), jnp.float32),
    ],
    compiler_params=pltpu.CompilerParams(needs_layout_passes=False),
)
def _sc_loss(preds, tgts, out, pv, tv, hist, red, cb, outv,
             sh_hist, sh_g, sh_tp, sh_tn, sh_acc):
    c = lax.axis_index("c")
    s = lax.axis_index("s")
    lane_off = lax.iota(jnp.int32, L) * (4 * NBINS)
    ones = jnp.ones((L,), jnp.float32)
    zeros = jnp.zeros((L,), jnp.float32)
    acc = zeros

    for img_i in range(IPC):
        img = c * IPC + img_i
        base = s * CHUNK
        pltpu.sync_copy(preds.at[img, pl.ds(base, CHUNK)], pv)
        pltpu.sync_copy(tgts.at[img, pl.ds(base, CHUNK)], tv)

        def _clr(i, carry):
            hist[pl.ds(i * (4 * L), L)] = zeros
            hist[pl.ds(i * (4 * L) + L, L)] = zeros
            hist[pl.ds(i * (4 * L) + 2 * L, L)] = zeros
            hist[pl.ds(i * (4 * L) + 3 * L, L)] = zeros
            return carry
        lax.fori_loop(0, HW // (4 * L), _clr, 0)

        def _elem(i, gacc):
            logit = pv[pl.ds(i * L, L)]
            g = tv[pl.ds(i * L, L)]
            gf = g.astype(jnp.float32)
            e = 1.0 - logit * (2.0 * gf - 1.0)
            m = e > 0.0
            b = jnp.clip((e * SCALE).astype(jnp.int32), 0, NBINS - 1)
            idx = lane_off + g * NBINS + b
            plsc.addupdate_scatter(hist, [idx], ones, mask=m)
            plsc.addupdate_scatter(hist, [idx + 2 * NBINS], e, mask=m)
            return gacc + gf
        gacc = lax.fori_loop(0, CHUNK // L, _elem, zeros)

        def _lred(i, carry):
            sp = i // (BLK // L)
            r = i - sp * (BLK // L)
            p = r // (BR // L)
            vj = r - p * (BR // L)
            src = p * NBINS + sp * BR + vj * L
            v = zeros
            for lane in range(L):
                v = v + hist[pl.ds(lane * (4 * NBINS) + src, L)]
            red[pl.ds(i * L, L)] = v
            return carry
        lax.fori_loop(0, RW // L, _lred, 0)

        for sp in range(NS):
            pltpu.sync_copy(red.at[pl.ds(sp * BLK, BLK)],
                            sh_hist.at[pl.ds(sp * (NS * BLK) + s * BLK, BLK)])
        outv[...] = gacc
        pltpu.sync_copy(outv, sh_g.at[pl.ds(s * L, L)])
        plsc.subcore_barrier()

        pltpu.sync_copy(sh_g, red.at[pl.ds(0, NS * L)])
        gv = zeros
        for t in range(NS):
            gv = gv + red[pl.ds(t * L, L)]
        G = jnp.sum(gv)

        pltpu.sync_copy(sh_hist.at[pl.ds(s * (NS * BLK), NS * BLK)], red)
        for o in range(0, BLK, L):
            v = zeros
            for t in range(NS):
                v = v + red[pl.ds(t * BLK + o, L)]
            cb[pl.ds(o, L)] = v

        tp = zeros
        tn = zeros
        for vj in range(BR // L):
            tn = tn + cb[pl.ds(vj * L, L)]
            tp = tp + cb[pl.ds(BR + vj * L, L)]
        tp_l = jnp.sum(tp)
        tn_l = jnp.sum(tn)
        outv[...] = zeros + tp_l
        pltpu.sync_copy(outv, sh_tp.at[pl.ds(s * L, L)])
        outv[...] = zeros + tn_l
        pltpu.sync_copy(outv, sh_tn.at[pl.ds(s * L, L)])
        plsc.subcore_barrier()

        pltpu.sync_copy(sh_tp, red.at[pl.ds(0, NS * L)])
        pltpu.sync_copy(sh_tn, red.at[pl.ds(NS * L, NS * L)])
        par = zeros
        nar = zeros
        for t in range(NS):
            above = jnp.int32(t) > s
            par = par + jnp.where(above, red[pl.ds(t * L, L)], zeros)
            nar = nar + jnp.where(above, red[pl.ds(NS * L + t * L, L)], zeros)

        carry_p = jnp.float32(0.0)
        carry_n = jnp.float32(0.0)
        for vj in range(BR // L):
            nv = cb[pl.ds(vj * L, L)]
            pvv = cb[pl.ds(BR + vj * L, L)]
            snv = cb[pl.ds(2 * BR + vj * L, L)]
            spv = cb[pl.ds(3 * BR + vj * L, L)]
            cps = carry_p + plsc.cumsum(pvv)
            cns = carry_n + plsc.cumsum(nv)
            carry_p = carry_p + jnp.sum(pvv)
            carry_n = carry_n + jnp.sum(nv)
            pa = par + (tp_l - cps)
            na = nar + (tn_l - cns)
            inv1 = 1.0 / (G + na)
            inv2 = 1.0 / (G + na + nv)
            tpos = spv * inv1
            tneg = snv * (G - pa - pvv) * (inv1 - inv2) / jnp.maximum(nv, 1.0)
            acc = acc + tpos + tneg
        plsc.subcore_barrier()

    outv[...] = acc
    pltpu.sync_copy(outv, sh_acc.at[pl.ds(s * L, L)])
    plsc.subcore_barrier()

    @pl.when(s == jnp.int32(0))
    def _():
        pltpu.sync_copy(sh_acc, red.at[pl.ds(0, NS * L)])
        tot = jnp.zeros((L,), jnp.float32)
        for t in range(NS):
            tot = tot + red[pl.ds(t * L, L)]
        outv[...] = jnp.zeros((L,), jnp.float32) + (jnp.sum(tot) * (1.0 / B))
        pltpu.sync_copy(outv, out.at[c])


def kernel(preds, targets):
    p = preds.reshape(B, N)
    t = targets.astype(jnp.int32).reshape(B, N)
    out = _sc_loss(p, t)
    return (out[0, 0] + out[1, 0]).reshape(())


# speedup vs baseline: 17.9502x; 1.2084x over previous
import functools

import jax
import jax.numpy as jnp
from jax import lax
from jax.experimental import pallas as pl
from jax.experimental.pallas import tpu as pltpu
from jax.experimental.pallas import tpu_sc as plsc

NC = 2
NS = 16
L = 16
B = 8
N = 512 * 512
IPC = B // NC
CHUNK = N // NS
NBINS = 512
EMAX = 16.0
SCALE = NBINS / EMAX
HW = L * 4 * NBINS
RW = 4 * NBINS
BR = NBINS // NS
BLK = 4 * BR
UNROLL = 4

_mesh = plsc.VectorSubcoreMesh(
    core_axis_name="c", subcore_axis_name="s", num_cores=NC, num_subcores=NS)


@functools.partial(
    pl.kernel,
    out_type=jax.ShapeDtypeStruct((NC, L), jnp.float32),
    mesh=_mesh,
    scratch_types=[
        pltpu.VMEM((2, CHUNK), jnp.float32),
        pltpu.VMEM((2, CHUNK), jnp.int32),
        pltpu.VMEM((HW,), jnp.float32),
        pltpu.VMEM((RW,), jnp.float32),
        pltpu.VMEM((BLK,), jnp.float32),
        pltpu.VMEM((L,), jnp.float32),
        pltpu.VMEM_SHARED((NS * BLK * NS,), jnp.float32),
        pltpu.VMEM_SHARED((NS * L,), jnp.float32),
        pltpu.VMEM_SHARED((NS * L,), jnp.float32),
        pltpu.VMEM_SHARED((NS * L,), jnp.float32),
        pltpu.VMEM_SHARED((NS * L,), jnp.float32),
        pltpu.SemaphoreType.DMA,
        pltpu.SemaphoreType.DMA,
    ],
    compiler_params=pltpu.CompilerParams(needs_layout_passes=False),
)
def _sc_loss(preds, tgts, out, pv, tv, hist, red, cb, outv,
             sh_hist, sh_g, sh_tp, sh_tn, sh_acc, sem_in, sem_pub):
    c = lax.axis_index("c")
    s = lax.axis_index("s")
    lane_off = lax.iota(jnp.int32, L) * (4 * NBINS)
    ones = jnp.ones((L,), jnp.float32)
    zeros = jnp.zeros((L,), jnp.float32)
    acc = zeros
    base = s * CHUNK

    def _clr(i, carry):
        for u in range(4):
            hist[pl.ds(i * (4 * L) + u * L, L)] = zeros
        return carry
    lax.fori_loop(0, HW // (4 * L), _clr, 0)

    cp = pltpu.async_copy(preds.at[c * IPC, pl.ds(base, CHUNK)], pv.at[0], sem_in)
    ct = pltpu.async_copy(tgts.at[c * IPC, pl.ds(base, CHUNK)], tv.at[0], sem_in)

    for img_i in range(IPC):
        buf = img_i % 2
        cp.wait()
        ct.wait()
        if img_i + 1 < IPC:
            nxt = c * IPC + img_i + 1
            cp = pltpu.async_copy(
                preds.at[nxt, pl.ds(base, CHUNK)], pv.at[1 - buf], sem_in)
            ct = pltpu.async_copy(
                tgts.at[nxt, pl.ds(base, CHUNK)], tv.at[1 - buf], sem_in)

        def _elem(i, gacc):
            for u in range(UNROLL):
                o = (i * UNROLL + u) * L
                logit = pv[buf, pl.ds(o, L)]
                g = tv[buf, pl.ds(o, L)]
                gf = g.astype(jnp.float32)
                e = 1.0 - logit * (2.0 * gf - 1.0)
                m = e > 0.0
                bb = jnp.minimum((e * SCALE).astype(jnp.int32), NBINS - 1)
                idx = lane_off + g * NBINS + bb
                plsc.addupdate_scatter(hist, [idx], ones, mask=m)
                plsc.addupdate_scatter(hist, [idx + 2 * NBINS], e, mask=m)
                gacc = gacc + gf
            return gacc
        gacc = lax.fori_loop(0, CHUNK // (L * UNROLL), _elem, zeros)

        def _lred(i, carry):
            sp = i // (BLK // L)
            r = i - sp * (BLK // L)
            p = r // (BR // L)
            vj = r - p * (BR // L)
            src = p * NBINS + sp * BR + vj * L
            v = zeros
            for lane in range(L):
                v = v + hist[pl.ds(lane * (4 * NBINS) + src, L)]
                hist[pl.ds(lane * (4 * NBINS) + src, L)] = zeros
            red[pl.ds(i * L, L)] = v
            return carry
        lax.fori_loop(0, RW // L, _lred, 0)

        pubs = []
        for sp in range(NS):
            pubs.append(pltpu.async_copy(
                red.at[pl.ds(sp * BLK, BLK)],
                sh_hist.at[pl.ds(sp * (NS * BLK) + s * BLK, BLK)], sem_pub))
        outv[...] = gacc
        pltpu.sync_copy(outv, sh_g.at[pl.ds(s * L, L)])
        for d in pubs:
            d.wait()
        plsc.subcore_barrier()

        pltpu.sync_copy(sh_g, red.at[pl.ds(0, NS * L)])
        gv = zeros
        for t in range(NS):
            gv = gv + red[pl.ds(t * L, L)]
        G = jnp.sum(gv)

        pltpu.sync_copy(sh_hist.at[pl.ds(s * (NS * BLK), NS * BLK)], red)
        for o in range(0, BLK, L):
            v = zeros
            for t in range(NS):
                v = v + red[pl.ds(t * BLK + o, L)]
            cb[pl.ds(o, L)] = v

        tp = zeros
        tn = zeros
        for vj in range(BR // L):
            tn = tn + cb[pl.ds(vj * L, L)]
            tp = tp + cb[pl.ds(BR + vj * L, L)]
        tp_l = jnp.sum(tp)
        tn_l = jnp.sum(tn)
        outv[...] = zeros + tp_l
        pltpu.sync_copy(outv, sh_tp.at[pl.ds(s * L, L)])
        outv[...] = zeros + tn_l
        pltpu.sync_copy(outv, sh_tn.at[pl.ds(s * L, L)])
        plsc.subcore_barrier()

        pltpu.sync_copy(sh_tp, red.at[pl.ds(0, NS * L)])
        pltpu.sync_copy(sh_tn, red.at[pl.ds(NS * L, NS * L)])
        par = zeros
        nar = zeros
        for t in range(NS):
            above = jnp.int32(t) > s
            par = par + jnp.where(above, red[pl.ds(t * L, L)], zeros)
            nar = nar + jnp.where(above, red[pl.ds(NS * L + t * L, L)], zeros)

        carry_p = jnp.float32(0.0)
        carry_n = jnp.float32(0.0)
        for vj in range(BR // L):
            nv = cb[pl.ds(vj * L, L)]
            pvv = cb[pl.ds(BR + vj * L, L)]
            snv = cb[pl.ds(2 * BR + vj * L, L)]
            spv = cb[pl.ds(3 * BR + vj * L, L)]
            cps = carry_p + plsc.cumsum(pvv)
            cns = carry_n + plsc.cumsum(nv)
            carry_p = carry_p + jnp.sum(pvv)
            carry_n = carry_n + jnp.sum(nv)
            pa = par + (tp_l - cps)
            na = nar + (tn_l - cns)
            inv1 = 1.0 / (G + na)
            inv2 = 1.0 / (G + na + nv)
            tpos = spv * inv1
            tneg = snv * (G - pa - pvv) * (inv1 - inv2) / jnp.maximum(nv, 1.0)
            acc = acc + tpos + tneg
        plsc.subcore_barrier()

    outv[...] = acc
    pltpu.sync_copy(outv, sh_acc.at[pl.ds(s * L, L)])
    plsc.subcore_barrier()

    @pl.when(s == jnp.int32(0))
    def _():
        pltpu.sync_copy(sh_acc, red.at[pl.ds(0, NS * L)])
        tot = jnp.zeros((L,), jnp.float32)
        for t in range(NS):
            tot = tot + red[pl.ds(t * L, L)]
        outv[...] = jnp.zeros((L,), jnp.float32) + (jnp.sum(tot) * (1.0 / B))
        pltpu.sync_copy(outv, out.at[c])


def kernel(preds, targets):
    p = preds.reshape(B, N)
    t = targets.astype(jnp.int32).reshape(B, N)
    out = _sc_loss(p, t)
    return (out[0, 0] + out[1, 0]).reshape(())


# speedup vs baseline: 34.1715x; 1.9037x over previous
import functools

import jax
import jax.numpy as jnp
from jax import lax
from jax.experimental import pallas as pl
from jax.experimental.pallas import tpu as pltpu
from jax.experimental.pallas import tpu_sc as plsc

NC = 2
NS = 16
L = 16
B = 8
N = 512 * 512
IPC = B // NC
CHUNK = N // NS
NBINS = 512
EMAX = 16.0
SCALE = NBINS / EMAX
HW = L * 4 * NBINS
RW = 4 * NBINS
BR = NBINS // NS
BLK = 4 * BR
UNROLL = 4

_mesh = plsc.VectorSubcoreMesh(
    core_axis_name="c", subcore_axis_name="s", num_cores=NC, num_subcores=NS)


@functools.partial(
    pl.kernel,
    out_type=jax.ShapeDtypeStruct((NC, L), jnp.float32),
    mesh=_mesh,
    scratch_types=[
        pltpu.VMEM((2, CHUNK), jnp.float32),
        pltpu.VMEM((2, CHUNK), jnp.int32),
        pltpu.VMEM((HW,), jnp.float32),
        pltpu.VMEM((RW,), jnp.float32),
        pltpu.VMEM((BLK,), jnp.float32),
        pltpu.VMEM((L,), jnp.float32),
        pltpu.VMEM_SHARED((NS * BLK * NS,), jnp.float32),
        pltpu.VMEM_SHARED((NS * L,), jnp.float32),
        pltpu.VMEM_SHARED((NS * L,), jnp.float32),
        pltpu.VMEM_SHARED((NS * L,), jnp.float32),
        pltpu.VMEM_SHARED((NS * L,), jnp.float32),
        pltpu.SemaphoreType.DMA,
        pltpu.SemaphoreType.DMA,
    ],
    compiler_params=pltpu.CompilerParams(needs_layout_passes=False),
)
def _sc_loss(preds, tgts, out, pv, tv, hist, red, cb, outv,
             sh_hist, sh_g, sh_tp, sh_tn, sh_acc, sem_in, sem_pub):
    c = lax.axis_index("c")
    s = lax.axis_index("s")
    lane_off = lax.iota(jnp.int32, L) * (4 * NBINS)
    ones = jnp.ones((L,), jnp.float32)
    zeros = jnp.zeros((L,), jnp.float32)
    acc = zeros
    base = s * CHUNK

    def _clr(i, carry):
        for u in range(4):
            hist[pl.ds(i * (4 * L) + u * L, L)] = zeros
        return carry
    lax.fori_loop(0, HW // (4 * L), _clr, 0)

    cp = pltpu.async_copy(preds.at[c * IPC, pl.ds(base, CHUNK)], pv.at[0], sem_in)
    ct = pltpu.async_copy(tgts.at[c * IPC, pl.ds(base, CHUNK)], tv.at[0], sem_in)

    for img_i in range(IPC):
        buf = img_i % 2
        cp.wait()
        ct.wait()
        if img_i + 1 < IPC:
            nxt = c * IPC + img_i + 1
            cp = pltpu.async_copy(
                preds.at[nxt, pl.ds(base, CHUNK)], pv.at[1 - buf], sem_in)
            ct = pltpu.async_copy(
                tgts.at[nxt, pl.ds(base, CHUNK)], tv.at[1 - buf], sem_in)

        @plsc.parallel_loop(0, CHUNK // L, 1, unroll=UNROLL, carry=zeros)
        def gacc(i, gacc):
            o = i * L
            logit = pv[buf, pl.ds(o, L)]
            g = tv[buf, pl.ds(o, L)]
            gf = g.astype(jnp.float32)
            e = 1.0 - logit * (2.0 * gf - 1.0)
            m = e > 0.0
            bb = jnp.minimum((e * SCALE).astype(jnp.int32), NBINS - 1)
            idx = lane_off + g * NBINS + bb
            plsc.addupdate_scatter(hist, [idx], ones, mask=m)
            plsc.addupdate_scatter(hist, [idx + 2 * NBINS], e, mask=m)
            return gacc + gf

        @plsc.parallel_loop(0, RW // L, 1, unroll=2)
        def _(i):
            sp = i // (BLK // L)
            r = i - sp * (BLK // L)
            p = r // (BR // L)
            vj = r - p * (BR // L)
            src = p * NBINS + sp * BR + vj * L
            vs = [hist[pl.ds(lane * (4 * NBINS) + src, L)] for lane in range(L)]
            while len(vs) > 1:
                vs = [a + b for a, b in zip(vs[::2], vs[1::2])]
            for lane in range(L):
                hist[pl.ds(lane * (4 * NBINS) + src, L)] = zeros
            red[pl.ds(i * L, L)] = vs[0]

        pubs = []
        for sp in range(NS):
            pubs.append(pltpu.async_copy(
                red.at[pl.ds(sp * BLK, BLK)],
                sh_hist.at[pl.ds(sp * (NS * BLK) + s * BLK, BLK)], sem_pub))
        outv[...] = gacc
        pltpu.sync_copy(outv, sh_g.at[pl.ds(s * L, L)])
        for d in pubs:
            d.wait()
        plsc.subcore_barrier()

        pltpu.sync_copy(sh_g, red.at[pl.ds(0, NS * L)])
        gvs = [red[pl.ds(t * L, L)] for t in range(NS)]
        while len(gvs) > 1:
            gvs = [a + b for a, b in zip(gvs[::2], gvs[1::2])]
        G = jnp.sum(gvs[0])

        pltpu.sync_copy(sh_hist.at[pl.ds(s * (NS * BLK), NS * BLK)], red)
        for o in range(0, BLK, L):
            vs = [red[pl.ds(t * BLK + o, L)] for t in range(NS)]
            while len(vs) > 1:
                vs = [a + b for a, b in zip(vs[::2], vs[1::2])]
            cb[pl.ds(o, L)] = vs[0]

        tp = zeros
        tn = zeros
        for vj in range(BR // L):
            tn = tn + cb[pl.ds(vj * L, L)]
            tp = tp + cb[pl.ds(BR + vj * L, L)]
        tp_l = jnp.sum(tp)
        tn_l = jnp.sum(tn)
        outv[...] = zeros + tp_l
        pltpu.sync_copy(outv, sh_tp.at[pl.ds(s * L, L)])
        outv[...] = zeros + tn_l
        pltpu.sync_copy(outv, sh_tn.at[pl.ds(s * L, L)])
        plsc.subcore_barrier()

        pltpu.sync_copy(sh_tp, red.at[pl.ds(0, NS * L)])
        pltpu.sync_copy(sh_tn, red.at[pl.ds(NS * L, NS * L)])
        par = zeros
        nar = zeros
        for t in range(NS):
            above = jnp.int32(t) > s
            par = par + jnp.where(above, red[pl.ds(t * L, L)], zeros)
            nar = nar + jnp.where(above, red[pl.ds(NS * L + t * L, L)], zeros)

        carry_p = jnp.float32(0.0)
        carry_n = jnp.float32(0.0)
        for vj in range(BR // L):
            nv = cb[pl.ds(vj * L, L)]
            pvv = cb[pl.ds(BR + vj * L, L)]
            snv = cb[pl.ds(2 * BR + vj * L, L)]
            spv = cb[pl.ds(3 * BR + vj * L, L)]
            cps = carry_p + plsc.cumsum(pvv)
            cns = carry_n + plsc.cumsum(nv)
            carry_p = carry_p + jnp.sum(pvv)
            carry_n = carry_n + jnp.sum(nv)
            pa = par + (tp_l - cps)
            na = nar + (tn_l - cns)
            inv1 = 1.0 / (G + na)
            inv2 = 1.0 / (G + na + nv)
            tpos = spv * inv1
            tneg = snv * (G - pa - pvv) * (inv1 - inv2) / jnp.maximum(nv, 1.0)
            acc = acc + tpos + tneg
        plsc.subcore_barrier()

    outv[...] = acc
    pltpu.sync_copy(outv, sh_acc.at[pl.ds(s * L, L)])
    plsc.subcore_barrier()

    @pl.when(s == jnp.int32(0))
    def _():
        pltpu.sync_copy(sh_acc, red.at[pl.ds(0, NS * L)])
        tot = jnp.zeros((L,), jnp.float32)
        for t in range(NS):
            tot = tot + red[pl.ds(t * L, L)]
        outv[...] = jnp.zeros((L,), jnp.float32) + (jnp.sum(tot) * (1.0 / B))
        pltpu.sync_copy(outv, out.at[c])


def kernel(preds, targets):
    p = preds.reshape(B, N)
    t = targets.astype(jnp.int32).reshape(B, N)
    out = _sc_loss(p, t)
    return (out[0, 0] + out[1, 0]).reshape(())


# speedup vs baseline: 48.6069x; 1.4224x over previous
import functools

import jax
import jax.numpy as jnp
from jax import lax
from jax.experimental import pallas as pl
from jax.experimental.pallas import tpu as pltpu
from jax.experimental.pallas import tpu_sc as plsc

NC = 2
NS = 16
L = 16
B = 8
N = 512 * 512
IPC = B // NC
CHUNK = N // NS
NBINS = 512
EMAX = 16.0
SCALE = NBINS / EMAX
HW = L * 4 * NBINS
RW = 4 * NBINS
BR = NBINS // NS
BLK = 4 * BR
UNROLL = 4

_mesh = plsc.VectorSubcoreMesh(
    core_axis_name="c", subcore_axis_name="s", num_cores=NC, num_subcores=NS)


@functools.partial(
    pl.kernel,
    out_type=jax.ShapeDtypeStruct((NC, L), jnp.float32),
    mesh=_mesh,
    scratch_types=[
        pltpu.VMEM((2, 32, 512), jnp.float32),
        pltpu.VMEM((2, 32, 512), jnp.int32),
        pltpu.VMEM((HW,), jnp.float32),
        pltpu.VMEM((RW,), jnp.float32),
        pltpu.VMEM((BLK,), jnp.float32),
        pltpu.VMEM((L,), jnp.float32),
        pltpu.VMEM_SHARED((NS * BLK * NS,), jnp.float32),
        pltpu.VMEM_SHARED((NS * L,), jnp.float32),
        pltpu.VMEM_SHARED((NS * L,), jnp.float32),
        pltpu.VMEM_SHARED((NS * L,), jnp.float32),
        pltpu.VMEM_SHARED((NS * L,), jnp.float32),
        pltpu.SemaphoreType.DMA,
        pltpu.SemaphoreType.DMA,
    ],
    compiler_params=pltpu.CompilerParams(
        needs_layout_passes=False, use_tc_tiling_on_sc=True),
)
def _sc_loss(preds, tgts, out, pv, tv, hist, red, cb, outv,
             sh_hist, sh_g, sh_tp, sh_tn, sh_acc, sem_in, sem_pub):
    c = lax.axis_index("c")
    s = lax.axis_index("s")
    lane_off = lax.iota(jnp.int32, L) * (4 * NBINS)
    ones = jnp.ones((L,), jnp.float32)
    zeros = jnp.zeros((L,), jnp.float32)
    acc = zeros
    base = s * CHUNK

    def _clr(i, carry):
        for u in range(4):
            hist[pl.ds(i * (4 * L) + u * L, L)] = zeros
        return carry
    lax.fori_loop(0, HW // (4 * L), _clr, 0)

    rbase = s * 32
    cp = pltpu.async_copy(
        preds.at[c * IPC, pl.ds(rbase, 32), :], pv.at[0], sem_in)
    ct = pltpu.async_copy(
        tgts.at[c * IPC, pl.ds(rbase, 32), :], tv.at[0], sem_in)

    for img_i in range(IPC):
        buf = img_i % 2
        cp.wait()
        ct.wait()
        if img_i + 1 < IPC:
            nxt = c * IPC + img_i + 1
            cp = pltpu.async_copy(
                preds.at[nxt, pl.ds(rbase, 32), :], pv.at[1 - buf], sem_in)
            ct = pltpu.async_copy(
                tgts.at[nxt, pl.ds(rbase, 32), :], tv.at[1 - buf], sem_in)

        @plsc.parallel_loop(0, CHUNK // L, 1, unroll=UNROLL, carry=zeros)
        def gacc(i, gacc):
            r = i // 32
            cc = (i - r * 32) * L
            logit = pv[buf, r, pl.ds(cc, L)]
            g = tv[buf, r, pl.ds(cc, L)]
            gf = g.astype(jnp.float32)
            e = 1.0 - logit * (2.0 * gf - 1.0)
            m = e > 0.0
            bb = jnp.minimum((e * SCALE).astype(jnp.int32), NBINS - 1)
            idx = lane_off + g * NBINS + bb
            plsc.addupdate_scatter(hist, [idx], ones, mask=m)
            plsc.addupdate_scatter(hist, [idx + 2 * NBINS], e, mask=m)
            return gacc + gf

        @plsc.parallel_loop(0, RW // L, 1, unroll=2)
        def _(i):
            sp = i // (BLK // L)
            r = i - sp * (BLK // L)
            p = r // (BR // L)
            vj = r - p * (BR // L)
            src = p * NBINS + sp * BR + vj * L
            vs = [hist[pl.ds(lane * (4 * NBINS) + src, L)] for lane in range(L)]
            while len(vs) > 1:
                vs = [a + b for a, b in zip(vs[::2], vs[1::2])]
            for lane in range(L):
                hist[pl.ds(lane * (4 * NBINS) + src, L)] = zeros
            red[pl.ds(i * L, L)] = vs[0]

        pubs = []
        for sp in range(NS):
            pubs.append(pltpu.async_copy(
                red.at[pl.ds(sp * BLK, BLK)],
                sh_hist.at[pl.ds(sp * (NS * BLK) + s * BLK, BLK)], sem_pub))
        outv[...] = gacc
        pltpu.sync_copy(outv, sh_g.at[pl.ds(s * L, L)])
        for d in pubs:
            d.wait()
        plsc.subcore_barrier()

        pltpu.sync_copy(sh_g, red.at[pl.ds(0, NS * L)])
        gvs = [red[pl.ds(t * L, L)] for t in range(NS)]
        while len(gvs) > 1:
            gvs = [a + b for a, b in zip(gvs[::2], gvs[1::2])]
        G = jnp.sum(gvs[0])

        pltpu.sync_copy(sh_hist.at[pl.ds(s * (NS * BLK), NS * BLK)], red)
        for o in range(0, BLK, L):
            vs = [red[pl.ds(t * BLK + o, L)] for t in range(NS)]
            while len(vs) > 1:
                vs = [a + b for a, b in zip(vs[::2], vs[1::2])]
            cb[pl.ds(o, L)] = vs[0]

        tp = zeros
        tn = zeros
        for vj in range(BR // L):
            tn = tn + cb[pl.ds(vj * L, L)]
            tp = tp + cb[pl.ds(BR + vj * L, L)]
        tp_l = jnp.sum(tp)
        tn_l = jnp.sum(tn)
        outv[...] = zeros + tp_l
        pltpu.sync_copy(outv, sh_tp.at[pl.ds(s * L, L)])
        outv[...] = zeros + tn_l
        pltpu.sync_copy(outv, sh_tn.at[pl.ds(s * L, L)])
        plsc.subcore_barrier()

        pltpu.sync_copy(sh_tp, red.at[pl.ds(0, NS * L)])
        pltpu.sync_copy(sh_tn, red.at[pl.ds(NS * L, NS * L)])
        par = zeros
        nar = zeros
        for t in range(NS):
            above = jnp.int32(t) > s
            par = par + jnp.where(above, red[pl.ds(t * L, L)], zeros)
            nar = nar + jnp.where(above, red[pl.ds(NS * L + t * L, L)], zeros)

        carry_p = jnp.float32(0.0)
        carry_n = jnp.float32(0.0)
        for vj in range(BR // L):
            nv = cb[pl.ds(vj * L, L)]
            pvv = cb[pl.ds(BR + vj * L, L)]
            snv = cb[pl.ds(2 * BR + vj * L, L)]
            spv = cb[pl.ds(3 * BR + vj * L, L)]
            cps = carry_p + plsc.cumsum(pvv)
            cns = carry_n + plsc.cumsum(nv)
            carry_p = carry_p + jnp.sum(pvv)
            carry_n = carry_n + jnp.sum(nv)
            pa = par + (tp_l - cps)
            na = nar + (tn_l - cns)
            inv1 = 1.0 / (G + na)
            inv2 = 1.0 / (G + na + nv)
            tpos = spv * inv1
            tneg = snv * (G - pa - pvv) * (inv1 - inv2) / jnp.maximum(nv, 1.0)
            acc = acc + tpos + tneg
        plsc.subcore_barrier()

    outv[...] = acc
    pltpu.sync_copy(outv, sh_acc.at[pl.ds(s * L, L)])
    plsc.subcore_barrier()

    @pl.when(s == jnp.int32(0))
    def _():
        pltpu.sync_copy(sh_acc, red.at[pl.ds(0, NS * L)])
        tot = jnp.zeros((L,), jnp.float32)
        for t in range(NS):
            tot = tot + red[pl.ds(t * L, L)]
        outv[...] = jnp.zeros((L,), jnp.float32) + (jnp.sum(tot) * (1.0 / B))
        pltpu.sync_copy(outv, out.at[c])


def kernel(preds, targets):
    t = targets.astype(jnp.int32)
    out = _sc_loss(preds, t)
    return (out[0, 0] + out[1, 0]).reshape(())
